# Initial kernel scaffold; baseline (speedup 1.0000x reference)
import jax
import jax.numpy as jnp
from jax.experimental import pallas as pl


def kernel(x, edge_index, W1, b1, W2, b2):
    raise NotImplementedError("write your pallas kernel here")


# speedup vs baseline: 12.3185x; 12.3185x over previous
import functools

import jax
import jax.numpy as jnp
from jax import lax
from jax.experimental import pallas as pl
from jax.experimental.pallas import tpu as pltpu
from jax.experimental.pallas import tpu_sc as plsc

N = 10000
E = 320000
IN_C = 128
HID_C = 64
OUT_C = 128

NPAD = 10240
EPAD = 327680
RPT = NPAD // 16
EPT = EPAD // 32
CH = 128

_mesh = plsc.VectorSubcoreMesh(core_axis_name="c", subcore_axis_name="s")


@functools.partial(
    pl.kernel,
    out_type=jax.ShapeDtypeStruct((2, NPAD), jnp.float32),
    mesh=_mesh,
    scratch_types=[
        pltpu.VMEM((EPT,), jnp.int32),
        pltpu.VMEM((NPAD,), jnp.float32),
        pltpu.VMEM((16, RPT), jnp.float32),
        pltpu.VMEM((RPT,), jnp.float32),
        pltpu.VMEM_SHARED((16, NPAD), jnp.float32),
    ],
    compiler_params=pltpu.CompilerParams(needs_layout_passes=False),
)
def _deg_kernel(dst_hbm, zeros_hbm, deg_out, dst_v, hist, buf, res, hist_sh):
    c = lax.axis_index("c")
    s = lax.axis_index("s")
    pltpu.sync_copy(zeros_hbm, hist)
    pltpu.sync_copy(dst_hbm.at[c, s], dst_v)
    ones = jnp.full((16,), 1.0, jnp.float32)

    def hbody(i, carry):
        idx = dst_v[pl.ds(i * 16, 16)]
        plsc.addupdate_scatter(hist, [idx], ones)
        return carry

    lax.fori_loop(0, EPT // 16, hbody, 0)
    pltpu.sync_copy(hist, hist_sh.at[s])
    plsc.subcore_barrier()
    base = s * RPT
    for t in range(16):
        pltpu.sync_copy(hist_sh.at[t, pl.ds(base, RPT)], buf.at[t])

    def cbody(i, carry):
        col = i * 16
        a = buf[0, pl.ds(col, 16)]
        for t in range(1, 16):
            a = a + buf[t, pl.ds(col, 16)]
        res[pl.ds(col, 16)] = a
        return carry

    lax.fori_loop(0, RPT // 16, cbody, 0)
    pltpu.sync_copy(res, deg_out.at[c, pl.ds(base, RPT)])


def _make_mp(edge_split):
    n_chunks = (EPAD // 32 // CH) if edge_split else (EPAD // 16 // CH)
    hs_shape = (NPAD, HID_C) if edge_split else (2, NPAD, HID_C)
    idx_shape = (2, 16, n_chunks, CH) if edge_split else (16, n_chunks, CH)

    @functools.partial(
        pl.kernel,
        out_type=jax.ShapeDtypeStruct((2, NPAD, HID_C), jnp.float32),
        mesh=_mesh,
        scratch_types=[
            pltpu.VMEM((n_chunks, CH), jnp.int32),
            pltpu.VMEM((n_chunks, CH), jnp.int32),
            pltpu.VMEM((CH, HID_C), jnp.float32),
            pltpu.VMEM_SHARED((NPAD, HID_C), jnp.float32),
        ],
        compiler_params=pltpu.CompilerParams(use_tc_tiling_on_sc=False),
    )
    def mp(hs_hbm, src_hbm, dst_hbm, zeros_hbm, out_hbm,
           src_v, dst_v, rows_v, acc_sh):
        c = lax.axis_index("c")
        s = lax.axis_index("s")
        base = s * RPT
        if edge_split:
            pltpu.sync_copy(src_hbm.at[c, s], src_v)
            pltpu.sync_copy(dst_hbm.at[c, s], dst_v)
        else:
            pltpu.sync_copy(src_hbm.at[s], src_v)
            pltpu.sync_copy(dst_hbm.at[s], dst_v)
        pltpu.sync_copy(zeros_hbm, acc_sh.at[pl.ds(base, RPT)])
        plsc.subcore_barrier()

        def ebody(j, carry):
            if edge_split:
                pltpu.sync_copy(hs_hbm.at[src_v.at[j]], rows_v)
            else:
                pltpu.sync_copy(hs_hbm.at[c].at[src_v.at[j]], rows_v)
            pltpu.sync_copy(rows_v, acc_sh.at[dst_v.at[j]], add=True)
            return carry

        lax.fori_loop(0, n_chunks, ebody, 0)
        plsc.subcore_barrier()
        pltpu.sync_copy(acc_sh.at[pl.ds(base, RPT)],
                        out_hbm.at[c, pl.ds(base, RPT)])

    return mp, idx_shape


_mp_layer1, _IDX1_SHAPE = _make_mp(edge_split=True)
_mp_layer2, _IDX2_SHAPE = _make_mp(edge_split=False)


BLK = 512
GRID = NPAD // BLK


def _tc_b_body(x_ref, w1_ref, d0_ref, d1_ref, hs1_ref, dinv_ref):
    deg = d0_ref[...] + d1_ref[...] + 1.0
    dinv = lax.rsqrt(deg)
    h = jnp.dot(x_ref[...], w1_ref[...], preferred_element_type=jnp.float32)
    hs1_ref[...] = dinv * h
    dinv_ref[...] = dinv


_tc_b = pl.pallas_call(
    _tc_b_body,
    grid=(GRID,),
    in_specs=[
        pl.BlockSpec((BLK, IN_C), lambda i: (i, 0)),
        pl.BlockSpec((IN_C, HID_C), lambda i: (0, 0)),
        pl.BlockSpec((BLK, 1), lambda i: (i, 0)),
        pl.BlockSpec((BLK, 1), lambda i: (i, 0)),
    ],
    out_specs=[
        pl.BlockSpec((BLK, HID_C), lambda i: (i, 0)),
        pl.BlockSpec((BLK, 1), lambda i: (i, 0)),
    ],
    out_shape=[
        jax.ShapeDtypeStruct((NPAD, HID_C), jnp.float32),
        jax.ShapeDtypeStruct((NPAD, 1), jnp.float32),
    ],
)


def _tc_d_body(acc1_ref, hs1_ref, dinv_ref, b1_ref, w2_ref, b2_ref,
               hs2s_ref, outb_ref):
    dv = dinv_ref[...]
    h1 = jnp.maximum(
        dv * (acc1_ref[0] + acc1_ref[1] + hs1_ref[...]) + b1_ref[...], 0.0)
    h2 = jnp.dot(h1, w2_ref[...], preferred_element_type=jnp.float32)
    outb_ref[...] = h2 + b2_ref[...]
    hs2 = dv * h2
    hs2s_ref[0] = hs2[:, :HID_C]
    hs2s_ref[1] = hs2[:, HID_C:]


_tc_d = pl.pallas_call(
    _tc_d_body,
    grid=(GRID,),
    in_specs=[
        pl.BlockSpec((2, BLK, HID_C), lambda i: (0, i, 0)),
        pl.BlockSpec((BLK, HID_C), lambda i: (i, 0)),
        pl.BlockSpec((BLK, 1), lambda i: (i, 0)),
        pl.BlockSpec((1, HID_C), lambda i: (0, 0)),
        pl.BlockSpec((HID_C, OUT_C), lambda i: (0, 0)),
        pl.BlockSpec((1, OUT_C), lambda i: (0, 0)),
    ],
    out_specs=[
        pl.BlockSpec((2, BLK, HID_C), lambda i: (0, i, 0)),
        pl.BlockSpec((BLK, OUT_C), lambda i: (i, 0)),
    ],
    out_shape=[
        jax.ShapeDtypeStruct((2, NPAD, HID_C), jnp.float32),
        jax.ShapeDtypeStruct((NPAD, OUT_C), jnp.float32),
    ],
)


def _tc_f_body(acc2_ref, hs2s_ref, dinv_ref, b2_ref, outa_ref):
    dv = dinv_ref[...]
    left = dv * (acc2_ref[0] + hs2s_ref[0])
    right = dv * (acc2_ref[1] + hs2s_ref[1])
    outa_ref[...] = jnp.concatenate([left, right], axis=1) + b2_ref[...]


_tc_f = pl.pallas_call(
    _tc_f_body,
    grid=(GRID,),
    in_specs=[
        pl.BlockSpec((2, BLK, HID_C), lambda i: (0, i, 0)),
        pl.BlockSpec((2, BLK, HID_C), lambda i: (0, i, 0)),
        pl.BlockSpec((BLK, 1), lambda i: (i, 0)),
        pl.BlockSpec((1, OUT_C), lambda i: (0, 0)),
    ],
    out_specs=pl.BlockSpec((BLK, OUT_C), lambda i: (i, 0)),
    out_shape=jax.ShapeDtypeStruct((NPAD, OUT_C), jnp.float32),
)


@jax.jit
def kernel(x, edge_index, W1, b1, W2, b2):
    src = edge_index[0]
    dst = edge_index[1]
    pad_e = EPAD - E
    srcp = jnp.concatenate([src, jnp.zeros((pad_e,), jnp.int32)])
    dstp = jnp.concatenate([dst, jnp.full((pad_e,), NPAD - 1, jnp.int32)])

    dst_a = dstp.reshape(2, 16, EPT)
    src_1 = srcp.reshape(_IDX1_SHAPE)
    dst_1 = dstp.reshape(_IDX1_SHAPE)
    src_2 = srcp.reshape(_IDX2_SHAPE)
    dst_2 = dstp.reshape(_IDX2_SHAPE)

    xp = jnp.pad(x, ((0, NPAD - N), (0, 0)))
    zeros_h = jnp.zeros((NPAD,), jnp.float32)
    zeros_r = jnp.zeros((RPT, HID_C), jnp.float32)

    deg_part = _deg_kernel(dst_a, zeros_h)
    hs1, dinv = _tc_b(xp, W1, deg_part[0][:, None], deg_part[1][:, None])
    acc1 = _mp_layer1(hs1, src_1, dst_1, zeros_r)
    hs2s, out_b = _tc_d(acc1, hs1, dinv, b1[None, :], W2, b2[None, :])
    acc2 = _mp_layer2(hs2s, src_2, dst_2, zeros_r)
    out_a = _tc_f(acc2, hs2s, dinv, b2[None, :])
    return (out_a[:N], out_b[:N])


# speedup vs baseline: 14.9672x; 1.2150x over previous
import functools

import jax
import jax.numpy as jnp
from jax import lax
from jax.experimental import pallas as pl
from jax.experimental.pallas import tpu as pltpu
from jax.experimental.pallas import tpu_sc as plsc

N = 10000
E = 320000
IN_C = 128
HID_C = 64
OUT_C = 128

NPAD = 10240
EPAD = 327680
RPT = NPAD // 16
EPT = EPAD // 32
CH = 128

_mesh = plsc.VectorSubcoreMesh(core_axis_name="c", subcore_axis_name="s")


@functools.partial(
    pl.kernel,
    out_type=jax.ShapeDtypeStruct((2, NPAD), jnp.float32),
    mesh=_mesh,
    scratch_types=[
        pltpu.VMEM((EPT,), jnp.int32),
        pltpu.VMEM((NPAD,), jnp.float32),
        pltpu.VMEM((16, RPT), jnp.float32),
        pltpu.VMEM((RPT,), jnp.float32),
        pltpu.VMEM_SHARED((16, NPAD), jnp.float32),
    ],
    compiler_params=pltpu.CompilerParams(needs_layout_passes=False),
)
def _deg_kernel(dst_hbm, zeros_hbm, deg_out, dst_v, hist, buf, res, hist_sh):
    c = lax.axis_index("c")
    s = lax.axis_index("s")
    pltpu.sync_copy(zeros_hbm, hist)
    pltpu.sync_copy(dst_hbm.at[c, s], dst_v)
    ones = jnp.full((16,), 1.0, jnp.float32)

    def hbody(i, carry):
        idx = dst_v[pl.ds(i * 16, 16)]
        plsc.addupdate_scatter(hist, [idx], ones)
        return carry

    lax.fori_loop(0, EPT // 16, hbody, 0)
    pltpu.sync_copy(hist, hist_sh.at[s])
    plsc.subcore_barrier()
    base = s * RPT
    for t in range(16):
        pltpu.sync_copy(hist_sh.at[t, pl.ds(base, RPT)], buf.at[t])

    def cbody(i, carry):
        col = i * 16
        a = buf[0, pl.ds(col, 16)]
        for t in range(1, 16):
            a = a + buf[t, pl.ds(col, 16)]
        res[pl.ds(col, 16)] = a
        return carry

    lax.fori_loop(0, RPT // 16, cbody, 0)
    pltpu.sync_copy(res, deg_out.at[c, pl.ds(base, RPT)])


def _make_mp(edge_split):
    n_chunks = (EPAD // 32 // CH) if edge_split else (EPAD // 16 // CH)
    hs_shape = (NPAD, HID_C) if edge_split else (2, NPAD, HID_C)
    idx_shape = (2, 16, n_chunks, CH) if edge_split else (16, n_chunks, CH)

    R = 4
    G = 2
    assert n_chunks % R == 0

    @functools.partial(
        pl.kernel,
        out_type=jax.ShapeDtypeStruct((2, NPAD, HID_C), jnp.float32),
        mesh=_mesh,
        scratch_types=[
            pltpu.VMEM((n_chunks, CH), jnp.int32),
            pltpu.VMEM((n_chunks, CH), jnp.int32),
            pltpu.VMEM((R, CH, HID_C), jnp.float32),
            pltpu.VMEM_SHARED((NPAD, HID_C), jnp.float32),
        ] + [pltpu.SemaphoreType.DMA] * (2 * R),
        compiler_params=pltpu.CompilerParams(use_tc_tiling_on_sc=False),
    )
    def mp(hs_hbm, src_hbm, dst_hbm, zeros_hbm, out_hbm,
           src_v, dst_v, rows_v, acc_sh, *sems):
        sem_g = sems[:R]
        sem_s = sems[R:]
        c = lax.axis_index("c")
        s = lax.axis_index("s")
        base = s * RPT
        if edge_split:
            pltpu.sync_copy(src_hbm.at[c, s], src_v)
            pltpu.sync_copy(dst_hbm.at[c, s], dst_v)
        else:
            pltpu.sync_copy(src_hbm.at[s], src_v)
            pltpu.sync_copy(dst_hbm.at[s], dst_v)
        pltpu.sync_copy(zeros_hbm, acc_sh.at[pl.ds(base, RPT)])
        plsc.subcore_barrier()

        def gather_src(j):
            if edge_split:
                return hs_hbm.at[src_v.at[j]]
            return hs_hbm.at[c].at[src_v.at[j]]

        def start_gather(j, b):
            pltpu.async_copy(gather_src(j), rows_v.at[b], sem_g[b])

        for t in range(G):
            start_gather(t, t)

        def obody(i, carry):
            jbase = i * R
            for t in range(R):
                j = jbase + t
                bg = (t + G) % R
                jg = j + G

                @pl.when(jg - R >= 0)
                def _wait_scatter():
                    pltpu.make_async_copy(
                        rows_v.at[bg], acc_sh.at[dst_v.at[j]], sem_s[bg]).wait()

                @pl.when(jg < n_chunks)
                def _prefetch():
                    start_gather(jg, bg)

                pltpu.make_async_copy(
                    gather_src(j), rows_v.at[t], sem_g[t]).wait()
                pltpu.async_copy(
                    rows_v.at[t], acc_sh.at[dst_v.at[j]], sem_s[t], add=True)
            return carry

        lax.fori_loop(0, n_chunks // R, obody, 0)
        for t in range(R - G):
            j = n_chunks - (R - G) + t
            b = j % R
            pltpu.make_async_copy(
                rows_v.at[b], acc_sh.at[dst_v.at[n_chunks - 1]], sem_s[b]).wait()
        plsc.subcore_barrier()
        pltpu.sync_copy(acc_sh.at[pl.ds(base, RPT)],
                        out_hbm.at[c, pl.ds(base, RPT)])

    return mp, idx_shape


_mp_layer1, _IDX1_SHAPE = _make_mp(edge_split=True)
_mp_layer2, _IDX2_SHAPE = _make_mp(edge_split=False)


BLK = 512
GRID = NPAD // BLK


def _tc_b_body(x_ref, w1_ref, d0_ref, d1_ref, hs1_ref, dinv_ref):
    deg = d0_ref[...] + d1_ref[...] + 1.0
    dinv = lax.rsqrt(deg)
    h = jnp.dot(x_ref[...], w1_ref[...], preferred_element_type=jnp.float32)
    hs1_ref[...] = dinv * h
    dinv_ref[...] = dinv


_tc_b = pl.pallas_call(
    _tc_b_body,
    grid=(GRID,),
    in_specs=[
        pl.BlockSpec((BLK, IN_C), lambda i: (i, 0)),
        pl.BlockSpec((IN_C, HID_C), lambda i: (0, 0)),
        pl.BlockSpec((BLK, 1), lambda i: (i, 0)),
        pl.BlockSpec((BLK, 1), lambda i: (i, 0)),
    ],
    out_specs=[
        pl.BlockSpec((BLK, HID_C), lambda i: (i, 0)),
        pl.BlockSpec((BLK, 1), lambda i: (i, 0)),
    ],
    out_shape=[
        jax.ShapeDtypeStruct((NPAD, HID_C), jnp.float32),
        jax.ShapeDtypeStruct((NPAD, 1), jnp.float32),
    ],
)


def _tc_d_body(acc1_ref, hs1_ref, dinv_ref, b1_ref, w2_ref, b2_ref,
               hs2s_ref, outb_ref):
    dv = dinv_ref[...]
    h1 = jnp.maximum(
        dv * (acc1_ref[0] + acc1_ref[1] + hs1_ref[...]) + b1_ref[...], 0.0)
    h2 = jnp.dot(h1, w2_ref[...], preferred_element_type=jnp.float32)
    outb_ref[...] = h2 + b2_ref[...]
    hs2 = dv * h2
    hs2s_ref[0] = hs2[:, :HID_C]
    hs2s_ref[1] = hs2[:, HID_C:]


_tc_d = pl.pallas_call(
    _tc_d_body,
    grid=(GRID,),
    in_specs=[
        pl.BlockSpec((2, BLK, HID_C), lambda i: (0, i, 0)),
        pl.BlockSpec((BLK, HID_C), lambda i: (i, 0)),
        pl.BlockSpec((BLK, 1), lambda i: (i, 0)),
        pl.BlockSpec((1, HID_C), lambda i: (0, 0)),
        pl.BlockSpec((HID_C, OUT_C), lambda i: (0, 0)),
        pl.BlockSpec((1, OUT_C), lambda i: (0, 0)),
    ],
    out_specs=[
        pl.BlockSpec((2, BLK, HID_C), lambda i: (0, i, 0)),
        pl.BlockSpec((BLK, OUT_C), lambda i: (i, 0)),
    ],
    out_shape=[
        jax.ShapeDtypeStruct((2, NPAD, HID_C), jnp.float32),
        jax.ShapeDtypeStruct((NPAD, OUT_C), jnp.float32),
    ],
)


def _tc_f_body(acc2_ref, hs2s_ref, dinv_ref, b2_ref, outa_ref):
    dv = dinv_ref[...]
    left = dv * (acc2_ref[0] + hs2s_ref[0])
    right = dv * (acc2_ref[1] + hs2s_ref[1])
    outa_ref[...] = jnp.concatenate([left, right], axis=1) + b2_ref[...]


_tc_f = pl.pallas_call(
    _tc_f_body,
    grid=(GRID,),
    in_specs=[
        pl.BlockSpec((2, BLK, HID_C), lambda i: (0, i, 0)),
        pl.BlockSpec((2, BLK, HID_C), lambda i: (0, i, 0)),
        pl.BlockSpec((BLK, 1), lambda i: (i, 0)),
        pl.BlockSpec((1, OUT_C), lambda i: (0, 0)),
    ],
    out_specs=pl.BlockSpec((BLK, OUT_C), lambda i: (i, 0)),
    out_shape=jax.ShapeDtypeStruct((NPAD, OUT_C), jnp.float32),
)


@jax.jit
def kernel(x, edge_index, W1, b1, W2, b2):
    src = edge_index[0]
    dst = edge_index[1]
    pad_e = EPAD - E
    srcp = jnp.concatenate([src, jnp.zeros((pad_e,), jnp.int32)])
    pad_dst = N + (jnp.arange(pad_e, dtype=jnp.int32) % (NPAD - N))
    dstp = jnp.concatenate([dst, pad_dst])

    dst_a = dstp.reshape(2, 16, EPT)
    src_1 = srcp.reshape(_IDX1_SHAPE)
    dst_1 = dstp.reshape(_IDX1_SHAPE)
    src_2 = srcp.reshape(_IDX2_SHAPE)
    dst_2 = dstp.reshape(_IDX2_SHAPE)

    xp = jnp.pad(x, ((0, NPAD - N), (0, 0)))
    zeros_h = jnp.zeros((NPAD,), jnp.float32)
    zeros_r = jnp.zeros((RPT, HID_C), jnp.float32)

    deg_part = _deg_kernel(dst_a, zeros_h)
    hs1, dinv = _tc_b(xp, W1, deg_part[0][:, None], deg_part[1][:, None])
    acc1 = _mp_layer1(hs1, src_1, dst_1, zeros_r)
    hs2s, out_b = _tc_d(acc1, hs1, dinv, b1[None, :], W2, b2[None, :])
    acc2 = _mp_layer2(hs2s, src_2, dst_2, zeros_r)
    out_a = _tc_f(acc2, hs2s, dinv, b2[None, :])
    return (out_a[:N], out_b[:N])


# speedup vs baseline: 31.6985x; 2.1179x over previous
import functools

import jax
import jax.numpy as jnp
from jax import lax
from jax.experimental import pallas as pl
from jax.experimental.pallas import tpu as pltpu
from jax.experimental.pallas import tpu_sc as plsc

N = 10000
E = 320000
IN_C = 128
HID_C = 64
OUT_C = 128

NPAD = 10240
EPAD = 327680
RPT = NPAD // 16
EPT = EPAD // 32
CH = 128

_mesh = plsc.VectorSubcoreMesh(core_axis_name="c", subcore_axis_name="s")


@functools.partial(
    pl.kernel,
    out_type=jax.ShapeDtypeStruct((2, NPAD), jnp.float32),
    mesh=_mesh,
    scratch_types=[
        pltpu.VMEM((EPT,), jnp.int32),
        pltpu.VMEM((NPAD,), jnp.float32),
        pltpu.VMEM((16, RPT), jnp.float32),
        pltpu.VMEM((RPT,), jnp.float32),
        pltpu.VMEM_SHARED((16, NPAD), jnp.float32),
    ],
    compiler_params=pltpu.CompilerParams(needs_layout_passes=False),
)
def _deg_kernel(dst_hbm, zeros_hbm, deg_out, dst_v, hist, buf, res, hist_sh):
    c = lax.axis_index("c")
    s = lax.axis_index("s")
    pltpu.sync_copy(zeros_hbm, hist)
    pltpu.sync_copy(dst_hbm.at[c, s], dst_v)
    ones = jnp.full((16,), 1.0, jnp.float32)

    def hbody(i, carry):
        idx = dst_v[pl.ds(i * 16, 16)]
        plsc.addupdate_scatter(hist, [idx], ones)
        return carry

    lax.fori_loop(0, EPT // 16, hbody, 0)
    pltpu.sync_copy(hist, hist_sh.at[s])
    plsc.subcore_barrier()
    base = s * RPT
    for t in range(16):
        pltpu.sync_copy(hist_sh.at[t, pl.ds(base, RPT)], buf.at[t])

    def cbody(i, carry):
        col = i * 16
        a = buf[0, pl.ds(col, 16)]
        for t in range(1, 16):
            a = a + buf[t, pl.ds(col, 16)]
        res[pl.ds(col, 16)] = a
        return carry

    lax.fori_loop(0, RPT // 16, cbody, 0)
    pltpu.sync_copy(res, deg_out.at[c, pl.ds(base, RPT)])


def _make_mp(edge_split):
    n_chunks = (EPAD // 32 // CH) if edge_split else (EPAD // 16 // CH)
    hs_shape = (NPAD, HID_C) if edge_split else (2, NPAD, HID_C)
    idx_shape = (2, 16, n_chunks, CH) if edge_split else (16, n_chunks, CH)

    R = 4
    G = 2
    assert n_chunks % R == 0

    @functools.partial(
        pl.kernel,
        out_type=jax.ShapeDtypeStruct((2, NPAD, HID_C), jnp.float32),
        mesh=_mesh,
        scratch_types=[
            pltpu.VMEM((n_chunks, CH), jnp.int32),
            pltpu.VMEM((n_chunks, CH), jnp.int32),
            pltpu.VMEM((R, CH, HID_C), jnp.float32),
            pltpu.VMEM_SHARED((NPAD, HID_C), jnp.float32),
        ] + [pltpu.SemaphoreType.DMA] * (2 * R),
        compiler_params=pltpu.CompilerParams(use_tc_tiling_on_sc=False),
    )
    def mp(hs_hbm, src_hbm, dst_hbm, zeros_hbm, out_hbm,
           src_v, dst_v, rows_v, acc_sh, *sems):
        sem_g = sems[:R]
        sem_s = sems[R:]
        c = lax.axis_index("c")
        s = lax.axis_index("s")
        base = s * RPT
        if edge_split:
            pltpu.sync_copy(src_hbm.at[c, s], src_v)
            pltpu.sync_copy(dst_hbm.at[c, s], dst_v)
        else:
            pltpu.sync_copy(src_hbm.at[s], src_v)
            pltpu.sync_copy(dst_hbm.at[s], dst_v)
        pltpu.sync_copy(zeros_hbm, acc_sh.at[pl.ds(base, RPT)])
        plsc.subcore_barrier()

        def gather_src(j):
            if edge_split:
                return hs_hbm.at[src_v.at[j]]
            return hs_hbm.at[c].at[src_v.at[j]]

        def start_gather(j, b):
            pltpu.async_copy(gather_src(j), rows_v.at[b], sem_g[b])

        for t in range(G):
            start_gather(t, t)

        def obody(i, carry):
            jbase = i * R
            for t in range(R):
                j = jbase + t
                bg = (t + G) % R
                jg = j + G

                @pl.when(jg - R >= 0)
                def _wait_scatter():
                    pltpu.make_async_copy(
                        rows_v.at[bg], acc_sh.at[dst_v.at[j]], sem_s[bg]).wait()

                @pl.when(jg < n_chunks)
                def _prefetch():
                    start_gather(jg, bg)

                pltpu.make_async_copy(
                    gather_src(j), rows_v.at[t], sem_g[t]).wait()
                pltpu.async_copy(
                    rows_v.at[t], acc_sh.at[dst_v.at[j]], sem_s[t], add=True)
            return carry

        lax.fori_loop(0, n_chunks // R, obody, 0)
        for t in range(R - G):
            j = n_chunks - (R - G) + t
            b = j % R
            pltpu.make_async_copy(
                rows_v.at[b], acc_sh.at[dst_v.at[n_chunks - 1]], sem_s[b]).wait()
        plsc.subcore_barrier()
        pltpu.sync_copy(acc_sh.at[pl.ds(base, RPT)],
                        out_hbm.at[c, pl.ds(base, RPT)])

    return mp, idx_shape


_mp_layer1, _IDX1_SHAPE = _make_mp(edge_split=True)
_mp_layer2, _IDX2_SHAPE = _make_mp(edge_split=False)


BLK = 512
GRID = NPAD // BLK


def _tc_b_body(x_ref, w1_ref, d0_ref, d1_ref, hs1_ref, dinv_ref):
    deg = d0_ref[...] + d1_ref[...] + 1.0
    dinv = lax.rsqrt(deg)
    h = jnp.dot(x_ref[...], w1_ref[...], preferred_element_type=jnp.float32)
    hs1_ref[...] = dinv * h
    dinv_ref[...] = dinv


_tc_b = pl.pallas_call(
    _tc_b_body,
    grid=(GRID,),
    in_specs=[
        pl.BlockSpec((BLK, IN_C), lambda i: (i, 0)),
        pl.BlockSpec((IN_C, HID_C), lambda i: (0, 0)),
        pl.BlockSpec((BLK, 1), lambda i: (i, 0)),
        pl.BlockSpec((BLK, 1), lambda i: (i, 0)),
    ],
    out_specs=[
        pl.BlockSpec((BLK, HID_C), lambda i: (i, 0)),
        pl.BlockSpec((BLK, 1), lambda i: (i, 0)),
    ],
    out_shape=[
        jax.ShapeDtypeStruct((NPAD, HID_C), jnp.float32),
        jax.ShapeDtypeStruct((NPAD, 1), jnp.float32),
    ],
)


def _tc_d_body(acc1_ref, hs1_ref, dinv_ref, b1_ref, w2_ref, b2_ref,
               hs2s_ref, outb_ref):
    dv = dinv_ref[...]
    h1 = jnp.maximum(
        dv * (acc1_ref[0] + acc1_ref[1] + hs1_ref[...]) + b1_ref[...], 0.0)
    h2 = jnp.dot(h1, w2_ref[...], preferred_element_type=jnp.float32)
    outb_ref[...] = h2 + b2_ref[...]
    hs2 = dv * h2
    hs2s_ref[0] = hs2[:, :HID_C]
    hs2s_ref[1] = hs2[:, HID_C:]


_tc_d = pl.pallas_call(
    _tc_d_body,
    grid=(GRID,),
    in_specs=[
        pl.BlockSpec((2, BLK, HID_C), lambda i: (0, i, 0)),
        pl.BlockSpec((BLK, HID_C), lambda i: (i, 0)),
        pl.BlockSpec((BLK, 1), lambda i: (i, 0)),
        pl.BlockSpec((1, HID_C), lambda i: (0, 0)),
        pl.BlockSpec((HID_C, OUT_C), lambda i: (0, 0)),
        pl.BlockSpec((1, OUT_C), lambda i: (0, 0)),
    ],
    out_specs=[
        pl.BlockSpec((2, BLK, HID_C), lambda i: (0, i, 0)),
        pl.BlockSpec((BLK, OUT_C), lambda i: (i, 0)),
    ],
    out_shape=[
        jax.ShapeDtypeStruct((2, NPAD, HID_C), jnp.float32),
        jax.ShapeDtypeStruct((NPAD, OUT_C), jnp.float32),
    ],
)


def _tc_f_body(acc2_ref, hs2s_ref, dinv_ref, b2_ref, outa_ref):
    dv = dinv_ref[...]
    left = dv * (acc2_ref[0] + hs2s_ref[0])
    right = dv * (acc2_ref[1] + hs2s_ref[1])
    outa_ref[...] = jnp.concatenate([left, right], axis=1) + b2_ref[...]


_tc_f = pl.pallas_call(
    _tc_f_body,
    grid=(GRID,),
    in_specs=[
        pl.BlockSpec((2, BLK, HID_C), lambda i: (0, i, 0)),
        pl.BlockSpec((2, BLK, HID_C), lambda i: (0, i, 0)),
        pl.BlockSpec((BLK, 1), lambda i: (i, 0)),
        pl.BlockSpec((1, OUT_C), lambda i: (0, 0)),
    ],
    out_specs=pl.BlockSpec((BLK, OUT_C), lambda i: (i, 0)),
    out_shape=jax.ShapeDtypeStruct((NPAD, OUT_C), jnp.float32),
)


@jax.jit
def kernel(x, edge_index, W1, b1, W2, b2):
    src = edge_index[0]
    dst = edge_index[1]
    ppt = (EPAD - E) // 32
    pad_rows = N + jnp.arange(ppt, dtype=jnp.int32)
    srcp = jnp.concatenate(
        [src.reshape(32, E // 32),
         jnp.broadcast_to(pad_rows, (32, ppt))], axis=1)
    dstp = jnp.concatenate(
        [dst.reshape(32, E // 32),
         jnp.broadcast_to(pad_rows, (32, ppt))], axis=1)

    dst_a = dstp.reshape(2, 16, EPT)
    src_1 = srcp.reshape(_IDX1_SHAPE)
    dst_1 = dstp.reshape(_IDX1_SHAPE)
    src_2 = srcp.reshape(_IDX2_SHAPE)
    dst_2 = dstp.reshape(_IDX2_SHAPE)

    xp = jnp.pad(x, ((0, NPAD - N), (0, 0)))
    zeros_h = jnp.zeros((NPAD,), jnp.float32)
    zeros_r = jnp.zeros((RPT, HID_C), jnp.float32)

    deg_part = _deg_kernel(dst_a, zeros_h)
    hs1, dinv = _tc_b(xp, W1, deg_part[0][:, None], deg_part[1][:, None])
    acc1 = _mp_layer1(hs1, src_1, dst_1, zeros_r)
    hs2s, out_b = _tc_d(acc1, hs1, dinv, b1[None, :], W2, b2[None, :])
    acc2 = _mp_layer2(hs2s, src_2, dst_2, zeros_r)
    out_a = _tc_f(acc2, hs2s, dinv, b2[None, :])
    return (out_a[:N], out_b[:N])
